# baseline (device time: 12404 ns/iter reference)
import jax
import jax.numpy as jnp
from jax import lax
from jax.experimental import pallas as pl
from jax.experimental.pallas import tpu as pltpu

N_DEV = 4
N_HALF = 2


def kernel(t, W):
    m, k = t.shape
    _, n = W.shape
    q = m // N_DEV
    hq = q // N_HALF

    def body(t_ref, w_ref, out_ref, src_buf,
             rs_comm, rs_send, rs_recv, ag_send, ag_recv):
        my = lax.axis_index("i")

        src_buf[...] = t_ref[...].astype(jnp.bfloat16)
        w_bf16 = w_ref[...].astype(jnp.bfloat16)

        barrier_sem = pltpu.get_barrier_semaphore()
        for off in range(1, N_DEV):
            peer = (my + off) % N_DEV
            pl.semaphore_signal(
                barrier_sem, inc=1,
                device_id=(peer,), device_id_type=pl.DeviceIdType.MESH,
            )
        pl.semaphore_wait(barrier_sem, N_DEV - 1)

        rs_rdmas = {}
        for h in range(N_HALF):
            for off in (2, 1, 3):
                peer = (my + off) % N_DEV
                rdma = pltpu.make_async_remote_copy(
                    src_ref=src_buf.at[pl.ds(peer * q + h * hq, hq)],
                    dst_ref=rs_comm.at[off - 1, h],
                    send_sem=rs_send.at[off - 1, h],
                    recv_sem=rs_recv.at[off - 1, h],
                    device_id=(peer,),
                    device_id_type=pl.DeviceIdType.MESH,
                )
                rdma.start()
                rs_rdmas[(off - 1, h)] = rdma

        ag_rdmas = {}
        for h in range(N_HALF):
            acc = t_ref[pl.ds(my * q + h * hq, hq)]
            for j in (0, 2, 1):
                rs_rdmas[(j, h)].wait_recv()
                acc = acc + rs_comm[j, h].astype(jnp.float32)
            y = lax.dot_general(
                acc.astype(jnp.bfloat16), w_bf16,
                (((1,), (0,)), ((), ())),
                preferred_element_type=jnp.float32,
            )
            out_ref[pl.ds(my * q + h * hq, hq)] = y.astype(jnp.bfloat16)
            for off in (2, 1, 3):
                peer = (my + off) % N_DEV
                rdma = pltpu.make_async_remote_copy(
                    src_ref=out_ref.at[pl.ds(my * q + h * hq, hq)],
                    dst_ref=out_ref.at[pl.ds(my * q + h * hq, hq)],
                    send_sem=ag_send.at[off - 1, h],
                    recv_sem=ag_recv.at[off - 1, h],
                    device_id=(peer,),
                    device_id_type=pl.DeviceIdType.MESH,
                )
                rdma.start()
                ag_rdmas[(off - 1, h)] = rdma

        for j in range(N_DEV - 1):
            for h in range(N_HALF):
                ag_rdmas[(j, h)].wait_recv()

        for key in rs_rdmas:
            rs_rdmas[key].wait_send()
        for key in ag_rdmas:
            ag_rdmas[key].wait_send()

    return pl.pallas_call(
        body,
        out_shape=jax.ShapeDtypeStruct((m, n), jnp.bfloat16),
        in_specs=[
            pl.BlockSpec(memory_space=pltpu.VMEM),
            pl.BlockSpec(memory_space=pltpu.VMEM),
        ],
        out_specs=pl.BlockSpec(memory_space=pltpu.VMEM),
        scratch_shapes=[
            pltpu.VMEM((m, k), jnp.bfloat16),
            pltpu.VMEM((N_DEV - 1, N_HALF, hq, k), jnp.bfloat16),
            pltpu.SemaphoreType.DMA((N_DEV - 1, N_HALF)),
            pltpu.SemaphoreType.DMA((N_DEV - 1, N_HALF)),
            pltpu.SemaphoreType.DMA((N_DEV - 1, N_HALF)),
            pltpu.SemaphoreType.DMA((N_DEV - 1, N_HALF)),
        ],
        compiler_params=pltpu.CompilerParams(collective_id=0),
    )(t, W)
